# baseline (speedup 1.0000x reference)
import functools

import jax
import jax.numpy as jnp
from jax import lax
from jax.experimental import pallas as pl
from jax.experimental.pallas import tpu as pltpu
from jax.experimental.pallas import tpu_sc as plsc

N = 10000
E = 320000
D = 128
DE = 16
H = 128
OUT = 100

HX = 144
SSCOL = 129

NC = 2
NS = 16
L = 16
NW = NC * NS
EPW = E // NW
K = 80
NCHUNK = EPW // K
SB = 25
NSB = NCHUNK // SB
NP = N
RPT = NP // NS

_mesh = plsc.VectorSubcoreMesh(core_axis_name="c", subcore_axis_name="s",
                               num_cores=NC, num_subcores=NS)


def _edge_body(hx_hbm, adst_hbm, src_hbm, dst_hbm, ae_hbm, ub_hbm, znum_hbm,
               num_out,
               num_sh, adst_v, ub_v, srcv, dstv, aev, wv,
               rows0, rows1, sg0, sg1, ss0, ss1):
  cid = lax.axis_index("c")
  tid = lax.axis_index("s")
  wid = tid * NC + cid

  pltpu.sync_copy(adst_hbm, adst_v)
  pltpu.sync_copy(ub_hbm, ub_v)

  r0 = tid * RPT
  pltpu.sync_copy(znum_hbm.at[pl.ds(r0, RPT)], num_sh.at[pl.ds(r0, RPT)])
  plsc.subcore_barrier()

  ubv = ub_v[...]
  rows = (rows0, rows1)
  sg = (sg0, sg1)
  ss = (ss0, ss1)
  row0 = wid * NCHUNK

  def start_gather(j, buf):
    pltpu.async_copy(hx_hbm.at[srcv.at[j]], rows[buf], sg[buf])

  def wait_gather(j, buf):
    pltpu.make_async_copy(hx_hbm.at[srcv.at[j]], rows[buf], sg[buf]).wait()

  def start_scatter(j, buf):
    pltpu.async_copy(rows[buf], num_sh.at[dstv.at[j]], ss[buf], add=True)

  def wait_scatter(j, buf):
    pltpu.make_async_copy(rows[buf], num_sh.at[dstv.at[j]], ss[buf]).wait()

  def compute_chunk(j, buf):
    rb = rows[buf]
    for g in range(K // L):
      sl = pl.ds(g * L, L)
      erow = g * L + lax.iota(jnp.int32, L)
      a1 = plsc.load_gather(rb, [erow, jnp.full((L,), H, jnp.int32)])
      a2 = plsc.load_gather(adst_v, [dstv[j, sl]])
      al = a1 + a2 + aev[j, sl]
      al = jnp.maximum(al, 0.2 * al)
      wv[sl] = jnp.exp(al - ubv)

    @plsc.parallel_loop(0, K, 1, unroll=4)
    def _(e):
      wrep = plsc.load_gather(wv, [jnp.full((L,), 0, jnp.int32) + e])
      for q in range(HX // L):
        s2 = pl.ds(q * L, L)
        rb[e, s2] = rb[e, s2] * wrep

  def super_body(s, _):
    base = row0 + s * SB
    pltpu.sync_copy(src_hbm.at[pl.ds(base, SB)], srcv)
    pltpu.sync_copy(dst_hbm.at[pl.ds(base, SB)], dstv)
    pltpu.sync_copy(ae_hbm.at[pl.ds(base, SB)], aev)

    start_gather(0, 0)
    wait_gather(0, 0)
    start_gather(1, 1)
    compute_chunk(0, 0)
    start_scatter(0, 0)

    def pair_body(t, _):
      c1 = 2 * t + 1
      wait_gather(c1, 1)
      wait_scatter(c1 - 1, 0)
      start_gather(c1 + 1, 0)
      compute_chunk(c1, 1)
      start_scatter(c1, 1)

      c2 = 2 * t + 2
      wait_gather(c2, 0)
      wait_scatter(c2 - 1, 1)
      start_gather(c2 + 1, 1)
      compute_chunk(c2, 0)
      start_scatter(c2, 0)
      return 0
    lax.fori_loop(0, (SB - 3) // 2, pair_body, 0)

    wait_gather(SB - 2, 1)
    wait_scatter(SB - 3, 0)
    start_gather(SB - 1, 0)
    compute_chunk(SB - 2, 1)
    start_scatter(SB - 2, 1)

    wait_gather(SB - 1, 0)
    compute_chunk(SB - 1, 0)
    start_scatter(SB - 1, 0)

    wait_scatter(SB - 2, 1)
    wait_scatter(SB - 1, 0)
    return 0

  lax.fori_loop(0, NSB, super_body, 0)
  plsc.subcore_barrier()

  pltpu.sync_copy(num_sh.at[pl.ds(r0, RPT)],
                  num_out.at[cid, pl.ds(r0, RPT)])


_edge = pl.kernel(
    _edge_body,
    out_type=[jax.ShapeDtypeStruct((NC, NP, HX), jnp.float32)],
    mesh=_mesh,
    scratch_types=[
        pltpu.VMEM_SHARED((NP, HX), jnp.float32),
        pltpu.VMEM((N,), jnp.float32),
        pltpu.VMEM((L,), jnp.float32),
        pltpu.VMEM((SB, K), jnp.int32),
        pltpu.VMEM((SB, K), jnp.int32),
        pltpu.VMEM((SB, K), jnp.float32),
        pltpu.VMEM((K,), jnp.float32),
        pltpu.VMEM((K, HX), jnp.float32),
        pltpu.VMEM((K, HX), jnp.float32),
        pltpu.SemaphoreType.DMA,
        pltpu.SemaphoreType.DMA,
        pltpu.SemaphoreType.DMA,
        pltpu.SemaphoreType.DMA,
    ],
    compiler_params=pltpu.CompilerParams(
        needs_layout_passes=False,
        use_tc_tiling_on_sc=False),
)

_HI = lax.Precision.HIGHEST


def _wblk(wev):
  cidx = lax.broadcasted_iota(jnp.int32, (8 * DE, 8), 0)
  kidx = lax.broadcasted_iota(jnp.int32, (8 * DE, 8), 1)
  sel = (cidx >> 4) == kidx
  wevfull = jnp.zeros((8 * DE, 8), jnp.float32)
  for j in range(DE):
    wevfull = wevfull + wev[0, j] * jnp.where((cidx & 15) == j, 1.0, 0.0)
  return jnp.where(sel, wevfull, 0.0)


_EB = 4000


def _packed_ae(wev, ea8):
  kidx = lax.broadcasted_iota(jnp.int32, (8, 8 * DE), 0)
  cidx = lax.broadcasted_iota(jnp.int32, (8, 8 * DE), 1)
  sel = (cidx >> 4) == kidx
  wevfull = jnp.zeros((8, 8 * DE), jnp.float32)
  for j in range(DE):
    wevfull = wevfull + wev[0, j] * jnp.where((cidx & 15) == j, 1.0, 0.0)
  b = jnp.where(sel, wevfull, 0.0)
  return lax.dot_general(b, ea8, (((1,), (1,)), ((), ())),
                         preferred_element_type=jnp.float32, precision=_HI)


def _ae_body(wev1_ref, wev2_ref, ea8_ref, aet1_ref, aet2_ref):
  ea8 = ea8_ref[...]
  aet1_ref[...] = _packed_ae(wev1_ref[...], ea8)
  aet2_ref[...] = _packed_ae(wev2_ref[...], ea8)


_ae = pl.pallas_call(
    _ae_body,
    grid=(E // 8 // _EB,),
    in_specs=[
        pl.BlockSpec((1, DE), lambda i: (0, 0)),
        pl.BlockSpec((1, DE), lambda i: (0, 0)),
        pl.BlockSpec((_EB, 8 * DE), lambda i: (i, 0)),
    ],
    out_specs=[
        pl.BlockSpec((8, _EB), lambda i: (i, 0)),
        pl.BlockSpec((8, _EB), lambda i: (i, 0)),
    ],
    out_shape=[
        jax.ShapeDtypeStruct((8 * (E // 8 // _EB), _EB), jnp.float32),
        jax.ShapeDtypeStruct((8 * (E // 8 // _EB), _EB), jnp.float32),
    ],
)


def _prep_body(x_ref, w1_ref, as1_ref, ad1_ref, we1_ref, ae1_ref, we2_ref,
               ae2_ref, hx_ref, adst_ref, ub_ref, wev1_ref, wev2_ref):
  h = jnp.dot(x_ref[...], w1_ref[...], preferred_element_type=jnp.float32,
              precision=_HI)
  asrc = jnp.dot(h, as1_ref[...], preferred_element_type=jnp.float32,
                 precision=_HI)
  adst = jnp.dot(h, ad1_ref[...], preferred_element_type=jnp.float32,
                 precision=_HI)
  hx_ref[...] = jnp.concatenate(
      [h, asrc, jnp.ones((N, HX - H - 1), jnp.float32)], axis=1)
  adst_ref[...] = adst
  ub = jnp.max(asrc) + jnp.max(adst)
  ub_ref[...] = jnp.full((1, 1), jnp.maximum(ub, 0.2 * ub))
  wev1_ref[...] = jnp.sum(we1_ref[...] * ae1_ref[...], axis=1)[None, :]
  wev2_ref[...] = jnp.sum(we2_ref[...] * ae2_ref[...], axis=1)[None, :]


_prep = pl.pallas_call(
    _prep_body,
    out_shape=[
        jax.ShapeDtypeStruct((N, HX), jnp.float32),
        jax.ShapeDtypeStruct((N, 1), jnp.float32),
        jax.ShapeDtypeStruct((1, 1), jnp.float32),
        jax.ShapeDtypeStruct((1, DE), jnp.float32),
        jax.ShapeDtypeStruct((1, DE), jnp.float32),
    ],
)

_NB = 2000
_NG = N // _NB


def _mid_body(num_ref, b1_ref, w2_ref, as2_ref, ad2_ref, hc1_ref,
              hx_ref, adst_ref, mx1_ref, mx2_ref, ub_ref, natt_ref):
  i = pl.program_id(0)
  acc = num_ref[0] + num_ref[1]
  num = acc[:, :H]
  ssum = acc[:, SSCOL:SSCOL + 1]
  h1 = jax.nn.relu(num / (ssum + 1e-16) + b1_ref[...])
  h2m = jnp.dot(h1, w2_ref[...], preferred_element_type=jnp.float32,
                precision=_HI)
  asrc = jnp.dot(h2m, as2_ref[...], preferred_element_type=jnp.float32,
                 precision=_HI)
  adst = jnp.dot(h2m, ad2_ref[...], preferred_element_type=jnp.float32,
                 precision=_HI)
  hx_ref[...] = jnp.concatenate(
      [h2m, asrc, jnp.ones((_NB, HX - H - 1), jnp.float32)], axis=1)
  adst_ref[...] = adst
  natt_ref[...] = jnp.dot(h1, hc1_ref[...], preferred_element_type=jnp.float32,
                          precision=_HI)

  @pl.when(i == 0)
  def _():
    mx1_ref[...] = jnp.full((1, 1), -jnp.inf)
    mx2_ref[...] = jnp.full((1, 1), -jnp.inf)

  mx1_ref[...] = jnp.maximum(mx1_ref[...], jnp.max(asrc))
  mx2_ref[...] = jnp.maximum(mx2_ref[...], jnp.max(adst))

  @pl.when(i == _NG - 1)
  def _():
    ub = mx1_ref[0, 0] + mx2_ref[0, 0]
    ub_ref[...] = jnp.full((1, 1), jnp.maximum(ub, 0.2 * ub))


_mid = pl.pallas_call(
    _mid_body,
    grid=(_NG,),
    in_specs=[
        pl.BlockSpec((NC, _NB, HX), lambda i: (0, i, 0)),
        pl.BlockSpec((1, H), lambda i: (0, 0)),
        pl.BlockSpec((H, H), lambda i: (0, 0)),
        pl.BlockSpec((H, 1), lambda i: (0, 0)),
        pl.BlockSpec((H, 1), lambda i: (0, 0)),
        pl.BlockSpec((H, 1), lambda i: (0, 0)),
    ],
    out_specs=[
        pl.BlockSpec((_NB, HX), lambda i: (i, 0)),
        pl.BlockSpec((_NB, 1), lambda i: (i, 0)),
        pl.BlockSpec((1, 1), lambda i: (0, 0)),
        pl.BlockSpec((1, 1), lambda i: (0, 0)),
        pl.BlockSpec((1, 1), lambda i: (0, 0)),
        pl.BlockSpec((_NB, 1), lambda i: (i, 0)),
    ],
    out_shape=[
        jax.ShapeDtypeStruct((N, HX), jnp.float32),
        jax.ShapeDtypeStruct((N, 1), jnp.float32),
        jax.ShapeDtypeStruct((1, 1), jnp.float32),
        jax.ShapeDtypeStruct((1, 1), jnp.float32),
        jax.ShapeDtypeStruct((1, 1), jnp.float32),
        jax.ShapeDtypeStruct((N, 1), jnp.float32),
    ],
)


def _post_body(num_ref, b2_ref, fcw_ref, fcb_ref, hc2_ref,
               natt1_ref, out_ref, natt_ref):
  acc = num_ref[0] + num_ref[1]
  num = acc[:, :H]
  ssum = acc[:, SSCOL:SSCOL + 1]
  h2 = jax.nn.relu(num / (ssum + 1e-16) + b2_ref[...])
  out_ref[...] = jnp.dot(h2, fcw_ref[...], preferred_element_type=jnp.float32,
                         precision=_HI) + fcb_ref[...]
  natt_ref[...] = natt1_ref[...] + jnp.dot(
      h2, hc2_ref[...], preferred_element_type=jnp.float32, precision=_HI)


_post = pl.pallas_call(
    _post_body,
    grid=(_NG,),
    in_specs=[
        pl.BlockSpec((NC, _NB, HX), lambda i: (0, i, 0)),
        pl.BlockSpec((1, H), lambda i: (0, 0)),
        pl.BlockSpec((H, OUT), lambda i: (0, 0)),
        pl.BlockSpec((1, OUT), lambda i: (0, 0)),
        pl.BlockSpec((H, 1), lambda i: (0, 0)),
        pl.BlockSpec((_NB, 1), lambda i: (i, 0)),
    ],
    out_specs=[
        pl.BlockSpec((_NB, OUT), lambda i: (i, 0)),
        pl.BlockSpec((_NB, 1), lambda i: (i, 0)),
    ],
    out_shape=[
        jax.ShapeDtypeStruct((N, OUT), jnp.float32),
        jax.ShapeDtypeStruct((N, 1), jnp.float32),
    ],
)


def kernel(x, edge_index, edge_attr, W1, as1, ad1, We1, ae1, b1, W2, as2,
           ad2, We2, ae2, b2, hc1, hc2, fcW, fcb):
  src2 = edge_index[0].reshape(E // K, K)
  dst2 = edge_index[1].reshape(E // K, K)
  znum = jnp.zeros((NP, HX), jnp.float32)

  ea8 = edge_attr.reshape(E // 8, 8 * DE)
  hx1, adst1, ub1, wev1, wev2 = _prep(
      x, W1, as1.reshape(D, 1), ad1.reshape(D, 1), We1, ae1.reshape(1, H),
      We2, ae2.reshape(1, H))
  aet1, aet2 = _ae(wev1, wev2, ea8)
  nblk = E // 8 // _EB
  ae1e = aet1.reshape(nblk, 8, _EB).transpose(0, 2, 1).reshape(E // K, K)
  ae2e = aet2.reshape(nblk, 8, _EB).transpose(0, 2, 1).reshape(E // K, K)

  num1 = _edge(hx1, adst1.reshape(N), src2, dst2, ae1e,
               jnp.full((L,), ub1[0, 0], jnp.float32), znum)[0]

  hx2, adst2, _mx1, _mx2, ub2, natt1 = _mid(
      num1, b1.reshape(1, H), W2, as2.reshape(H, 1), ad2.reshape(H, 1), hc1)

  num2 = _edge(hx2, adst2.reshape(N), src2, dst2, ae2e,
               jnp.full((L,), ub2[0, 0], jnp.float32), znum)[0]

  out, natt = _post(
      num2, b2.reshape(1, H), fcW, fcb.reshape(1, OUT), hc2, natt1)

  return out, natt.reshape(N)


# speedup vs baseline: 1.0627x; 1.0627x over previous
import jax
import jax.numpy as jnp
from jax import lax
from jax.experimental import pallas as pl
from jax.experimental.pallas import tpu as pltpu
from jax.experimental.pallas import tpu_sc as plsc

N = 10000
E = 320000
D = 128
DE = 16
H = 128
OUT = 100

HX = 144
SSCOL = 129

NC = 2
NS = 16
L = 16
NW = NC * NS
EPW = E // NW
K = 80
NCHUNK = EPW // K
SB = 25
NSB = NCHUNK // SB
NP = N
RPT = NP // NS

_mesh = plsc.VectorSubcoreMesh(core_axis_name="c", subcore_axis_name="s",
                               num_cores=NC, num_subcores=NS)


def _edge_body(hx_hbm, adst_hbm, src_hbm, dst_hbm, ae_hbm, ub_hbm, znum_hbm,
               num_out,
               num_sh, adst_v, ub_v, srcv, dstv, aev, wv,
               rows0, rows1, sg0, sg1, ss0, ss1):
  cid = lax.axis_index("c")
  tid = lax.axis_index("s")
  wid = tid * NC + cid

  pltpu.sync_copy(adst_hbm, adst_v)
  pltpu.sync_copy(ub_hbm, ub_v)

  r0 = tid * RPT
  pltpu.sync_copy(znum_hbm.at[pl.ds(r0, RPT)], num_sh.at[pl.ds(r0, RPT)])
  plsc.subcore_barrier()

  ubv = ub_v[...]
  rows = (rows0, rows1)
  sg = (sg0, sg1)
  ss = (ss0, ss1)
  row0 = wid * NCHUNK

  def start_gather(j, buf):
    pltpu.async_copy(hx_hbm.at[srcv.at[j]], rows[buf], sg[buf])

  def wait_gather(j, buf):
    pltpu.make_async_copy(hx_hbm.at[srcv.at[j]], rows[buf], sg[buf]).wait()

  def start_scatter(j, buf):
    pltpu.async_copy(rows[buf], num_sh.at[dstv.at[j]], ss[buf], add=True)

  def wait_scatter(j, buf):
    pltpu.make_async_copy(rows[buf], num_sh.at[dstv.at[j]], ss[buf]).wait()

  def compute_chunk(j, buf):
    rb = rows[buf]
    for g in range(K // L):
      sl = pl.ds(g * L, L)
      erow = g * L + lax.iota(jnp.int32, L)
      a1 = plsc.load_gather(rb, [erow, jnp.full((L,), H, jnp.int32)])
      a2 = plsc.load_gather(adst_v, [dstv[j, sl]])
      al = a1 + a2 + aev[j, sl]
      al = jnp.maximum(al, 0.2 * al)
      wv[sl] = jnp.exp(al - ubv)

    @plsc.parallel_loop(0, K, 1, unroll=2)
    def _(e):
      wrep = plsc.load_gather(wv, [jnp.full((L,), 0, jnp.int32) + e])
      for q in range(HX // L):
        s2 = pl.ds(q * L, L)
        rb[e, s2] = rb[e, s2] * wrep

  def super_body(s, _):
    base = row0 + s * SB
    pltpu.sync_copy(src_hbm.at[pl.ds(base, SB)], srcv)
    pltpu.sync_copy(dst_hbm.at[pl.ds(base, SB)], dstv)
    pltpu.sync_copy(ae_hbm.at[pl.ds(base, SB)], aev)

    start_gather(0, 0)
    wait_gather(0, 0)
    start_gather(1, 1)
    compute_chunk(0, 0)
    start_scatter(0, 0)

    def pair_body(t, _):
      c1 = 2 * t + 1
      wait_gather(c1, 1)
      wait_scatter(c1 - 1, 0)
      start_gather(c1 + 1, 0)
      compute_chunk(c1, 1)
      start_scatter(c1, 1)

      c2 = 2 * t + 2
      wait_gather(c2, 0)
      wait_scatter(c2 - 1, 1)
      start_gather(c2 + 1, 1)
      compute_chunk(c2, 0)
      start_scatter(c2, 0)
      return 0
    lax.fori_loop(0, (SB - 3) // 2, pair_body, 0)

    wait_gather(SB - 2, 1)
    wait_scatter(SB - 3, 0)
    start_gather(SB - 1, 0)
    compute_chunk(SB - 2, 1)
    start_scatter(SB - 2, 1)

    wait_gather(SB - 1, 0)
    compute_chunk(SB - 1, 0)
    start_scatter(SB - 1, 0)

    wait_scatter(SB - 2, 1)
    wait_scatter(SB - 1, 0)
    return 0

  lax.fori_loop(0, NSB, super_body, 0)
  plsc.subcore_barrier()

  pltpu.sync_copy(num_sh.at[pl.ds(r0, RPT)],
                  num_out.at[cid, pl.ds(r0, RPT)])


_edge = pl.kernel(
    _edge_body,
    out_type=[jax.ShapeDtypeStruct((NC, NP, HX), jnp.float32)],
    mesh=_mesh,
    scratch_types=[
        pltpu.VMEM_SHARED((NP, HX), jnp.float32),
        pltpu.VMEM((N,), jnp.float32),
        pltpu.VMEM((L,), jnp.float32),
        pltpu.VMEM((SB, K), jnp.int32),
        pltpu.VMEM((SB, K), jnp.int32),
        pltpu.VMEM((SB, K), jnp.float32),
        pltpu.VMEM((K,), jnp.float32),
        pltpu.VMEM((K, HX), jnp.float32),
        pltpu.VMEM((K, HX), jnp.float32),
        pltpu.SemaphoreType.DMA,
        pltpu.SemaphoreType.DMA,
        pltpu.SemaphoreType.DMA,
        pltpu.SemaphoreType.DMA,
    ],
    compiler_params=pltpu.CompilerParams(
        needs_layout_passes=False,
        use_tc_tiling_on_sc=False),
)

_HI = lax.Precision.DEFAULT


def _wblk(wev):
  cidx = lax.broadcasted_iota(jnp.int32, (8 * DE, 8), 0)
  kidx = lax.broadcasted_iota(jnp.int32, (8 * DE, 8), 1)
  sel = (cidx >> 4) == kidx
  wevfull = jnp.zeros((8 * DE, 8), jnp.float32)
  for j in range(DE):
    wevfull = wevfull + wev[0, j] * jnp.where((cidx & 15) == j, 1.0, 0.0)
  return jnp.where(sel, wevfull, 0.0)


_EB = 4000


def _packed_ae(wev, ea8):
  kidx = lax.broadcasted_iota(jnp.int32, (8, 8 * DE), 0)
  cidx = lax.broadcasted_iota(jnp.int32, (8, 8 * DE), 1)
  sel = (cidx >> 4) == kidx
  wevfull = jnp.zeros((8, 8 * DE), jnp.float32)
  for j in range(DE):
    wevfull = wevfull + wev[0, j] * jnp.where((cidx & 15) == j, 1.0, 0.0)
  b = jnp.where(sel, wevfull, 0.0)
  return lax.dot_general(b, ea8, (((1,), (1,)), ((), ())),
                         preferred_element_type=jnp.float32, precision=_HI)


def _ae_body(wev1_ref, wev2_ref, ea8_ref, aet1_ref, aet2_ref):
  ea8 = ea8_ref[...]
  aet1_ref[...] = _packed_ae(wev1_ref[...], ea8)
  aet2_ref[...] = _packed_ae(wev2_ref[...], ea8)


_ae = pl.pallas_call(
    _ae_body,
    grid=(E // 8 // _EB,),
    in_specs=[
        pl.BlockSpec((1, DE), lambda i: (0, 0)),
        pl.BlockSpec((1, DE), lambda i: (0, 0)),
        pl.BlockSpec((_EB, 8 * DE), lambda i: (i, 0)),
    ],
    out_specs=[
        pl.BlockSpec((8, _EB), lambda i: (i, 0)),
        pl.BlockSpec((8, _EB), lambda i: (i, 0)),
    ],
    out_shape=[
        jax.ShapeDtypeStruct((8 * (E // 8 // _EB), _EB), jnp.float32),
        jax.ShapeDtypeStruct((8 * (E // 8 // _EB), _EB), jnp.float32),
    ],
)


def _prep_body(x_ref, w1_ref, as1_ref, ad1_ref, we1_ref, ae1_ref, we2_ref,
               ae2_ref, hx_ref, adst_ref, ub_ref, wev1_ref, wev2_ref):
  h = jnp.dot(x_ref[...], w1_ref[...], preferred_element_type=jnp.float32,
              precision=_HI)
  asrc = jnp.dot(h, as1_ref[...], preferred_element_type=jnp.float32,
                 precision=_HI)
  adst = jnp.dot(h, ad1_ref[...], preferred_element_type=jnp.float32,
                 precision=_HI)
  hx_ref[...] = jnp.concatenate(
      [h, asrc, jnp.ones((N, HX - H - 1), jnp.float32)], axis=1)
  adst_ref[...] = adst
  ub = jnp.max(asrc) + jnp.max(adst)
  ub_ref[...] = jnp.full((1, 1), jnp.maximum(ub, 0.2 * ub))
  wev1_ref[...] = jnp.sum(we1_ref[...] * ae1_ref[...], axis=1)[None, :]
  wev2_ref[...] = jnp.sum(we2_ref[...] * ae2_ref[...], axis=1)[None, :]


_prep = pl.pallas_call(
    _prep_body,
    out_shape=[
        jax.ShapeDtypeStruct((N, HX), jnp.float32),
        jax.ShapeDtypeStruct((N, 1), jnp.float32),
        jax.ShapeDtypeStruct((1, 1), jnp.float32),
        jax.ShapeDtypeStruct((1, DE), jnp.float32),
        jax.ShapeDtypeStruct((1, DE), jnp.float32),
    ],
)

_NB = 2000
_NG = N // _NB


def _mid_body(num_ref, b1_ref, w2_ref, as2_ref, ad2_ref, hc1_ref,
              hx_ref, adst_ref, mx1_ref, mx2_ref, ub_ref, natt_ref):
  i = pl.program_id(0)
  acc = num_ref[0] + num_ref[1]
  num = acc[:, :H]
  ssum = acc[:, SSCOL:SSCOL + 1]
  h1 = jax.nn.relu(num / (ssum + 1e-16) + b1_ref[...])
  h2m = jnp.dot(h1, w2_ref[...], preferred_element_type=jnp.float32,
                precision=_HI)
  asrc = jnp.dot(h2m, as2_ref[...], preferred_element_type=jnp.float32,
                 precision=_HI)
  adst = jnp.dot(h2m, ad2_ref[...], preferred_element_type=jnp.float32,
                 precision=_HI)
  hx_ref[...] = jnp.concatenate(
      [h2m, asrc, jnp.ones((_NB, HX - H - 1), jnp.float32)], axis=1)
  adst_ref[...] = adst
  natt_ref[...] = jnp.dot(h1, hc1_ref[...], preferred_element_type=jnp.float32,
                          precision=_HI)

  @pl.when(i == 0)
  def _():
    mx1_ref[...] = jnp.full((1, 1), -jnp.inf)
    mx2_ref[...] = jnp.full((1, 1), -jnp.inf)

  mx1_ref[...] = jnp.maximum(mx1_ref[...], jnp.max(asrc))
  mx2_ref[...] = jnp.maximum(mx2_ref[...], jnp.max(adst))

  @pl.when(i == _NG - 1)
  def _():
    ub = mx1_ref[0, 0] + mx2_ref[0, 0]
    ub_ref[...] = jnp.full((1, 1), jnp.maximum(ub, 0.2 * ub))


_mid = pl.pallas_call(
    _mid_body,
    grid=(_NG,),
    in_specs=[
        pl.BlockSpec((NC, _NB, HX), lambda i: (0, i, 0)),
        pl.BlockSpec((1, H), lambda i: (0, 0)),
        pl.BlockSpec((H, H), lambda i: (0, 0)),
        pl.BlockSpec((H, 1), lambda i: (0, 0)),
        pl.BlockSpec((H, 1), lambda i: (0, 0)),
        pl.BlockSpec((H, 1), lambda i: (0, 0)),
    ],
    out_specs=[
        pl.BlockSpec((_NB, HX), lambda i: (i, 0)),
        pl.BlockSpec((_NB, 1), lambda i: (i, 0)),
        pl.BlockSpec((1, 1), lambda i: (0, 0)),
        pl.BlockSpec((1, 1), lambda i: (0, 0)),
        pl.BlockSpec((1, 1), lambda i: (0, 0)),
        pl.BlockSpec((_NB, 1), lambda i: (i, 0)),
    ],
    out_shape=[
        jax.ShapeDtypeStruct((N, HX), jnp.float32),
        jax.ShapeDtypeStruct((N, 1), jnp.float32),
        jax.ShapeDtypeStruct((1, 1), jnp.float32),
        jax.ShapeDtypeStruct((1, 1), jnp.float32),
        jax.ShapeDtypeStruct((1, 1), jnp.float32),
        jax.ShapeDtypeStruct((N, 1), jnp.float32),
    ],
)


def _post_body(num_ref, b2_ref, fcw_ref, fcb_ref, hc2_ref,
               natt1_ref, out_ref, natt_ref):
  acc = num_ref[0] + num_ref[1]
  num = acc[:, :H]
  ssum = acc[:, SSCOL:SSCOL + 1]
  h2 = jax.nn.relu(num / (ssum + 1e-16) + b2_ref[...])
  out_ref[...] = jnp.dot(h2, fcw_ref[...], preferred_element_type=jnp.float32,
                         precision=_HI) + fcb_ref[...]
  natt_ref[...] = natt1_ref[...] + jnp.dot(
      h2, hc2_ref[...], preferred_element_type=jnp.float32, precision=_HI)


_post = pl.pallas_call(
    _post_body,
    grid=(_NG,),
    in_specs=[
        pl.BlockSpec((NC, _NB, HX), lambda i: (0, i, 0)),
        pl.BlockSpec((1, H), lambda i: (0, 0)),
        pl.BlockSpec((H, OUT), lambda i: (0, 0)),
        pl.BlockSpec((1, OUT), lambda i: (0, 0)),
        pl.BlockSpec((H, 1), lambda i: (0, 0)),
        pl.BlockSpec((_NB, 1), lambda i: (i, 0)),
    ],
    out_specs=[
        pl.BlockSpec((_NB, OUT), lambda i: (i, 0)),
        pl.BlockSpec((_NB, 1), lambda i: (i, 0)),
    ],
    out_shape=[
        jax.ShapeDtypeStruct((N, OUT), jnp.float32),
        jax.ShapeDtypeStruct((N, 1), jnp.float32),
    ],
)


def kernel(x, edge_index, edge_attr, W1, as1, ad1, We1, ae1, b1, W2, as2,
           ad2, We2, ae2, b2, hc1, hc2, fcW, fcb):
  src2 = edge_index[0].reshape(E // K, K)
  dst2 = edge_index[1].reshape(E // K, K)
  znum = jnp.zeros((NP, HX), jnp.float32)

  ea8 = edge_attr.reshape(E // 8, 8 * DE)
  hx1, adst1, ub1, wev1, wev2 = _prep(
      x, W1, as1.reshape(D, 1), ad1.reshape(D, 1), We1, ae1.reshape(1, H),
      We2, ae2.reshape(1, H))
  aet1, aet2 = _ae(wev1, wev2, ea8)
  nblk = E // 8 // _EB
  ae1e = aet1.reshape(nblk, 8, _EB).transpose(0, 2, 1).reshape(E // K, K)
  ae2e = aet2.reshape(nblk, 8, _EB).transpose(0, 2, 1).reshape(E // K, K)

  num1 = _edge(hx1, adst1.reshape(N), src2, dst2, ae1e,
               jnp.full((L,), ub1[0, 0], jnp.float32), znum)[0]

  hx2, adst2, _mx1, _mx2, ub2, natt1 = _mid(
      num1, b1.reshape(1, H), W2, as2.reshape(H, 1), ad2.reshape(H, 1), hc1)

  num2 = _edge(hx2, adst2.reshape(N), src2, dst2, ae2e,
               jnp.full((L,), ub2[0, 0], jnp.float32), znum)[0]

  out, natt = _post(
      num2, b2.reshape(1, H), fcW, fcb.reshape(1, OUT), hc2, natt1)

  return out, natt.reshape(N)
